# baseline (speedup 1.0000x reference)
import functools

import jax
import jax.numpy as jnp
from jax import lax
from jax.experimental import pallas as pl
from jax.experimental.pallas import tpu as pltpu
from jax.experimental.pallas import tpu_sc as plsc
import numpy as np

_MU = 1024
_RS = float(1.0 / np.sqrt(np.float32(1.0 + 1e-5)))
_IDX_MASK = 0xFFF


def _knn_body(pu_ref, pd_ref, bdf_ref, bdl_ref,
              i0_ref, i1_ref, w0_ref, w1_ref, *, nd, chunk):
    inf = jnp.float32(jnp.inf)
    maxi = jnp.int32(2**31 - 1)
    nj = pd_ref.shape[0]

    bu_first = pu_ref[0, 6]
    bu_last = pu_ref[_MU - 1, 6]
    c_lo = jnp.int32(nj)
    c_hi = jnp.int32(0)
    for c in range(nj):
        a = jnp.logical_and(bdf_ref[0, c] <= bu_last,
                            bdl_ref[0, c] >= bu_first)
        c_lo = jnp.where(jnp.logical_and(a, c_lo == nj), jnp.int32(c), c_lo)
        c_hi = jnp.where(a, jnp.int32(c + 1), c_hi)

    pu = pu_ref[...]
    bu = pu_ref[:, 6:7]

    def body(c, carry):
        r0, r1 = carry
        pd = pd_ref[c]
        d2 = lax.dot_general(pu, pd, (((1,), (0,)), ((), ())),
                             preferred_element_type=jnp.float32,
                             precision=lax.Precision.HIGHEST)
        d2 = jnp.maximum(d2, 0.0)
        bd = pd_ref[c, 5:6, :]
        d2 = jnp.where(bu != bd, inf, d2)
        col = lax.broadcasted_iota(jnp.int32, (1, chunk), 1) + c * chunk
        key = (lax.bitcast_convert_type(d2, jnp.int32) & ~_IDX_MASK) | col
        k1 = jnp.min(key, axis=1, keepdims=True)
        keyb = jnp.where(key == k1, maxi, key)
        k2 = jnp.min(keyb, axis=1, keepdims=True)
        n0 = jnp.minimum(k1, r0)
        n1 = jnp.minimum(jnp.maximum(k1, r0), jnp.minimum(k2, r1))
        return n0, n1

    init = (jnp.full((_MU, 1), maxi, jnp.int32),
            jnp.full((_MU, 1), maxi, jnp.int32))
    k0, k1 = lax.fori_loop(c_lo, c_hi, body, init)

    i0_ref[...] = jnp.minimum(k0 & _IDX_MASK, nd - 1)
    i1_ref[...] = jnp.minimum(k1 & _IDX_MASK, nd - 1)
    d20 = lax.bitcast_convert_type(k0 & ~_IDX_MASK, jnp.float32)
    d21 = lax.bitcast_convert_type(k1 & ~_IDX_MASK, jnp.float32)
    w0_ref[...] = 1.0 / jnp.maximum(d20, 1e-16)
    w1_ref[...] = 1.0 / jnp.maximum(d21, 1e-16)


def _knn(pu8, pd3d, bdf, bdl, nd, chunk):
    nu = pu8.shape[0]
    nj = pd3d.shape[0]
    grid = (nu // _MU,)
    out1 = jax.ShapeDtypeStruct((nu, 1), jnp.int32)
    outf = jax.ShapeDtypeStruct((nu, 1), jnp.float32)
    return pl.pallas_call(
        functools.partial(_knn_body, nd=nd, chunk=chunk),
        grid=grid,
        in_specs=[
            pl.BlockSpec((_MU, 8), lambda t: (t, 0)),
            pl.BlockSpec((nj, 8, chunk), lambda t: (0, 0, 0)),
            pl.BlockSpec((1, nj), lambda t: (0, 0)),
            pl.BlockSpec((1, nj), lambda t: (0, 0)),
        ],
        out_specs=[
            pl.BlockSpec((_MU, 1), lambda t: (t, 0)),
            pl.BlockSpec((_MU, 1), lambda t: (t, 0)),
            pl.BlockSpec((_MU, 1), lambda t: (t, 0)),
            pl.BlockSpec((_MU, 1), lambda t: (t, 0)),
        ],
        out_shape=[out1, out1, outf, outf],
    )(pu8, pd3d, bdf, bdl)


def _gather_pairs(table, idx0, idx1):
    nd, d = table.shape
    nu = idx0.shape[0]
    nw = 32
    rows = nu // nw
    ch = rows
    while ch * d * 4 > 131072:
        ch //= 2
    mesh = plsc.VectorSubcoreMesh(core_axis_name="c", subcore_axis_name="s")

    @functools.partial(
        pl.kernel,
        out_type=(jax.ShapeDtypeStruct((nu, d), jnp.float32),
                  jax.ShapeDtypeStruct((nu, d), jnp.float32)),
        mesh=mesh,
        scratch_types=[
            pltpu.VMEM((ch,), jnp.int32),
            pltpu.VMEM((ch,), jnp.int32),
            pltpu.VMEM((ch, d), jnp.float32),
            pltpu.VMEM((ch, d), jnp.float32),
            pltpu.SemaphoreType.DMA,
            pltpu.SemaphoreType.DMA,
            pltpu.SemaphoreType.DMA,
            pltpu.SemaphoreType.DMA,
        ],
    )
    def gk(tab, i0h, i1h, o0h, o1h, i0v, i1v, rows0, rows1,
           sg0, sg1, sw0, sw1):
        wid = lax.axis_index("s") * 2 + lax.axis_index("c")
        w0 = w1 = None
        for t in range(rows // ch):
            base = wid * rows + t * ch
            pltpu.sync_copy(i0h.at[pl.ds(base, ch)], i0v)
            pltpu.sync_copy(i1h.at[pl.ds(base, ch)], i1v)
            if w0 is not None:
                w0.wait()
                w1.wait()
            g0 = pltpu.async_copy(tab.at[i0v], rows0, sg0)
            g1 = pltpu.async_copy(tab.at[i1v], rows1, sg1)
            g0.wait()
            w0 = pltpu.async_copy(rows0, o0h.at[pl.ds(base, ch)], sw0)
            g1.wait()
            w1 = pltpu.async_copy(rows1, o1h.at[pl.ds(base, ch)], sw1)
        w0.wait()
        w1.wait()

    return gk(table, idx0, idx1)


def _mlp_body(xu_ref, k0_ref, k1_ref, w0_ref, w1_ref,
              wa_ref, wb_ref, c0_ref, g0_ref, e0_ref,
              w1w_ref, c1_ref, g1_ref, e1_ref, o_ref):
    w0 = w0_ref[...]
    w1 = w1_ref[...]
    xi = (k0_ref[...] * w0 + k1_ref[...] * w1) * (1.0 / (w0 + w1))
    ya = lax.dot_general(xu_ref[...], wa_ref[...], (((1,), (1,)), ((), ())),
                         preferred_element_type=jnp.float32,
                         precision=lax.Precision.DEFAULT)
    yb = lax.dot_general(xi, wb_ref[...], (((1,), (1,)), ((), ())),
                         preferred_element_type=jnp.float32,
                         precision=lax.Precision.DEFAULT)
    s0 = g0_ref[...] * _RS
    t0 = c0_ref[...] * s0 + e0_ref[...]
    y = jnp.maximum((ya + yb) * s0 + t0, 0.0)
    y = lax.dot_general(y, w1w_ref[...], (((1,), (1,)), ((), ())),
                        preferred_element_type=jnp.float32,
                        precision=lax.Precision.DEFAULT)
    s1 = g1_ref[...] * _RS
    t1 = c1_ref[...] * s1 + e1_ref[...]
    o_ref[...] = jnp.maximum(y * s1 + t1, 0.0)


def _mlp(x_up, xk0, xk1, w0, w1, wa, wb, c0, g0, e0, w1w, c1, g1, e1):
    nu, cup = x_up.shape
    d = xk0.shape[1]
    ch2 = w1w.shape[0]
    grid = (nu // _MU,)
    full = lambda a: pl.BlockSpec(a.shape, lambda t: tuple(0 for _ in a.shape))
    return pl.pallas_call(
        _mlp_body,
        grid=grid,
        in_specs=[
            pl.BlockSpec((_MU, cup), lambda t: (t, 0)),
            pl.BlockSpec((_MU, d), lambda t: (t, 0)),
            pl.BlockSpec((_MU, d), lambda t: (t, 0)),
            pl.BlockSpec((_MU, 1), lambda t: (t, 0)),
            pl.BlockSpec((_MU, 1), lambda t: (t, 0)),
            full(wa), full(wb), full(c0), full(g0), full(e0),
            full(w1w), full(c1), full(g1), full(e1),
        ],
        out_specs=pl.BlockSpec((_MU, ch2), lambda t: (t, 0)),
        out_shape=jax.ShapeDtypeStruct((nu, ch2), jnp.float32),
    )(x_up, xk0, xk1, w0, w1, wa, wb, c0, g0, e0, w1w, c1, g1, e1)


def _stage(p_up, x_up, b_up, p_down, x_down, b_down,
           w_l0, c_l0, g_l0, e_l0, w_l1, c_l1, g_l1, e_l1, chunk):
    nu = p_up.shape[0]
    nd = p_down.shape[0]
    nj = nd // chunk
    cup = x_up.shape[1]
    zu = jnp.zeros((nu, 1), jnp.float32)
    pu8 = jnp.concatenate(
        [p_up * -2.0, jnp.sum(p_up * p_up, axis=1, keepdims=True),
         jnp.ones((nu, 1), jnp.float32), zu,
         b_up.astype(jnp.float32).reshape(nu, 1), zu], axis=1)
    bdow = b_down.astype(jnp.float32).reshape(1, nd)
    pd_ex = jnp.concatenate(
        [p_down.T, jnp.ones((1, nd), jnp.float32),
         jnp.sum(p_down * p_down, axis=1).reshape(1, nd),
         bdow, jnp.zeros((2, nd), jnp.float32)], axis=0)
    pd3d = pd_ex.reshape(8, nj, chunk).swapaxes(0, 1)
    bdf = bdow[0, ::chunk].reshape(1, nj)
    bdl = bdow[0, chunk - 1::chunk].reshape(1, nj)
    i0, i1, w0, w1 = _knn(pu8, pd3d, bdf, bdl, nd, chunk)
    xk0, xk1 = _gather_pairs(x_down, i0.reshape(nu), i1.reshape(nu))
    wa = w_l0[:, :cup]
    wb = w_l0[:, cup:]
    row = lambda v: v.reshape(1, -1)
    return _mlp(x_up, xk0, xk1, w0, w1, wa, wb,
                row(c_l0), row(g_l0), row(e_l0),
                w_l1, row(c_l1), row(g_l1), row(e_l1))


def kernel(p0, x0, b0, p1, x1, b1, p2, x2, b2, p3, x3, b3,
           W0_0, c0_0, g0_0, e0_0, W0_1, c0_1, g0_1, e0_1,
           W1_0, c1_0, g1_0, e1_0, W1_1, c1_1, g1_1, e1_1,
           W2_0, c2_0, g2_0, e2_0, W2_1, c2_1, g2_1, e2_1):
    x2n = _stage(p2, x2, b2, p3, x3, b3,
                 W2_0, c2_0, g2_0, e2_0, W2_1, c2_1, g2_1, e2_1, chunk=128)
    x1n = _stage(p1, x1, b1, p2, x2n, b2,
                 W1_0, c1_0, g1_0, e1_0, W1_1, c1_1, g1_1, e1_1, chunk=256)
    x0n = _stage(p0, x0, b0, p1, x1n, b1,
                 W0_0, c0_0, g0_0, e0_0, W0_1, c0_1, g0_1, e0_1, chunk=1024)
    return x0n


# speedup vs baseline: 1.3305x; 1.3305x over previous
import functools

import jax
import jax.numpy as jnp
from jax import lax
from jax.experimental import pallas as pl
from jax.experimental.pallas import tpu as pltpu
from jax.experimental.pallas import tpu_sc as plsc
import numpy as np

_MU = 1024
_RS = float(1.0 / np.sqrt(np.float32(1.0 + 1e-5)))
_IDX_MASK = 0xFFF


def _knn_body(pu_ref, pd_ref, bdf_ref, bdl_ref,
              i0_ref, i1_ref, w0_ref, w1_ref, *, nd, chunk):
    inf = jnp.float32(jnp.inf)
    maxi = jnp.int32(2**31 - 1)
    nj = pd_ref.shape[0]

    bu_first = pu_ref[0, 6]
    bu_last = pu_ref[_MU - 1, 6]
    c_lo = jnp.int32(nj)
    c_hi = jnp.int32(0)
    for c in range(nj):
        a = jnp.logical_and(bdf_ref[0, c] <= bu_last,
                            bdl_ref[0, c] >= bu_first)
        c_lo = jnp.where(jnp.logical_and(a, c_lo == nj), jnp.int32(c), c_lo)
        c_hi = jnp.where(a, jnp.int32(c + 1), c_hi)

    pu = pu_ref[...]
    bu = pu_ref[:, 6:7]

    def body(c, carry):
        r0, r1 = carry
        pd = pd_ref[c]
        d2 = lax.dot_general(pu, pd, (((1,), (0,)), ((), ())),
                             preferred_element_type=jnp.float32,
                             precision=lax.Precision.DEFAULT)
        d2 = jnp.maximum(d2, 0.0)
        bd = pd_ref[c, 5:6, :]
        d2 = jnp.where(bu != bd, inf, d2)
        col = lax.broadcasted_iota(jnp.int32, (1, chunk), 1) + c * chunk
        key = (lax.bitcast_convert_type(d2, jnp.int32) & ~_IDX_MASK) | col
        k1 = jnp.min(key, axis=1, keepdims=True)
        keyb = jnp.where(key == k1, maxi, key)
        k2 = jnp.min(keyb, axis=1, keepdims=True)
        n0 = jnp.minimum(k1, r0)
        n1 = jnp.minimum(jnp.maximum(k1, r0), jnp.minimum(k2, r1))
        return n0, n1

    init = (jnp.full((_MU, 1), maxi, jnp.int32),
            jnp.full((_MU, 1), maxi, jnp.int32))
    k0, k1 = lax.fori_loop(c_lo, c_hi, body, init)

    i0_ref[...] = jnp.minimum(k0 & _IDX_MASK, nd - 1)
    i1_ref[...] = jnp.minimum(k1 & _IDX_MASK, nd - 1)
    d20 = lax.bitcast_convert_type(k0 & ~_IDX_MASK, jnp.float32)
    d21 = lax.bitcast_convert_type(k1 & ~_IDX_MASK, jnp.float32)
    w0_ref[...] = 1.0 / jnp.maximum(d20, 1e-16)
    w1_ref[...] = 1.0 / jnp.maximum(d21, 1e-16)


def _knn(pu8, pd3d, bdf, bdl, nd, chunk):
    nu = pu8.shape[0]
    nj = pd3d.shape[0]
    grid = (nu // _MU,)
    out1 = jax.ShapeDtypeStruct((nu, 1), jnp.int32)
    outf = jax.ShapeDtypeStruct((nu, 1), jnp.float32)
    return pl.pallas_call(
        functools.partial(_knn_body, nd=nd, chunk=chunk),
        grid=grid,
        in_specs=[
            pl.BlockSpec((_MU, 8), lambda t: (t, 0)),
            pl.BlockSpec((nj, 8, chunk), lambda t: (0, 0, 0)),
            pl.BlockSpec((1, nj), lambda t: (0, 0)),
            pl.BlockSpec((1, nj), lambda t: (0, 0)),
        ],
        out_specs=[
            pl.BlockSpec((_MU, 1), lambda t: (t, 0)),
            pl.BlockSpec((_MU, 1), lambda t: (t, 0)),
            pl.BlockSpec((_MU, 1), lambda t: (t, 0)),
            pl.BlockSpec((_MU, 1), lambda t: (t, 0)),
        ],
        out_shape=[out1, out1, outf, outf],
    )(pu8, pd3d, bdf, bdl)


def _gather_pairs(table, idx0, idx1):
    nd, d = table.shape
    nu = idx0.shape[0]
    nw = 32
    rows = nu // nw
    ch = rows
    while ch * d * 4 > 131072:
        ch //= 2
    mesh = plsc.VectorSubcoreMesh(core_axis_name="c", subcore_axis_name="s")

    @functools.partial(
        pl.kernel,
        out_type=(jax.ShapeDtypeStruct((nu, d), jnp.float32),
                  jax.ShapeDtypeStruct((nu, d), jnp.float32)),
        mesh=mesh,
        scratch_types=[
            pltpu.VMEM((ch,), jnp.int32),
            pltpu.VMEM((ch,), jnp.int32),
            pltpu.VMEM((ch, d), jnp.float32),
            pltpu.VMEM((ch, d), jnp.float32),
            pltpu.SemaphoreType.DMA,
            pltpu.SemaphoreType.DMA,
            pltpu.SemaphoreType.DMA,
            pltpu.SemaphoreType.DMA,
        ],
    )
    def gk(tab, i0h, i1h, o0h, o1h, i0v, i1v, rows0, rows1,
           sg0, sg1, sw0, sw1):
        wid = lax.axis_index("s") * 2 + lax.axis_index("c")
        w0 = w1 = None
        for t in range(rows // ch):
            base = wid * rows + t * ch
            pltpu.sync_copy(i0h.at[pl.ds(base, ch)], i0v)
            pltpu.sync_copy(i1h.at[pl.ds(base, ch)], i1v)
            if w0 is not None:
                w0.wait()
                w1.wait()
            g0 = pltpu.async_copy(tab.at[i0v], rows0, sg0)
            g1 = pltpu.async_copy(tab.at[i1v], rows1, sg1)
            g0.wait()
            w0 = pltpu.async_copy(rows0, o0h.at[pl.ds(base, ch)], sw0)
            g1.wait()
            w1 = pltpu.async_copy(rows1, o1h.at[pl.ds(base, ch)], sw1)
        w0.wait()
        w1.wait()

    return gk(table, idx0, idx1)


def _mlp_body(xu_ref, k0_ref, k1_ref, w0_ref, w1_ref,
              wa_ref, wb_ref, c0_ref, g0_ref, e0_ref,
              w1w_ref, c1_ref, g1_ref, e1_ref, o_ref):
    w0 = w0_ref[...]
    w1 = w1_ref[...]
    xi = (k0_ref[...] * w0 + k1_ref[...] * w1) * (1.0 / (w0 + w1))
    ya = lax.dot_general(xu_ref[...], wa_ref[...], (((1,), (1,)), ((), ())),
                         preferred_element_type=jnp.float32,
                         precision=lax.Precision.DEFAULT)
    yb = lax.dot_general(xi, wb_ref[...], (((1,), (1,)), ((), ())),
                         preferred_element_type=jnp.float32,
                         precision=lax.Precision.DEFAULT)
    s0 = g0_ref[...] * _RS
    t0 = c0_ref[...] * s0 + e0_ref[...]
    y = jnp.maximum((ya + yb) * s0 + t0, 0.0)
    y = lax.dot_general(y, w1w_ref[...], (((1,), (1,)), ((), ())),
                        preferred_element_type=jnp.float32,
                        precision=lax.Precision.DEFAULT)
    s1 = g1_ref[...] * _RS
    t1 = c1_ref[...] * s1 + e1_ref[...]
    o_ref[...] = jnp.maximum(y * s1 + t1, 0.0)


def _mlp(x_up, xk0, xk1, w0, w1, wa, wb, c0, g0, e0, w1w, c1, g1, e1):
    nu, cup = x_up.shape
    d = xk0.shape[1]
    ch2 = w1w.shape[0]
    grid = (nu // _MU,)
    full = lambda a: pl.BlockSpec(a.shape, lambda t: tuple(0 for _ in a.shape))
    return pl.pallas_call(
        _mlp_body,
        grid=grid,
        in_specs=[
            pl.BlockSpec((_MU, cup), lambda t: (t, 0)),
            pl.BlockSpec((_MU, d), lambda t: (t, 0)),
            pl.BlockSpec((_MU, d), lambda t: (t, 0)),
            pl.BlockSpec((_MU, 1), lambda t: (t, 0)),
            pl.BlockSpec((_MU, 1), lambda t: (t, 0)),
            full(wa), full(wb), full(c0), full(g0), full(e0),
            full(w1w), full(c1), full(g1), full(e1),
        ],
        out_specs=pl.BlockSpec((_MU, ch2), lambda t: (t, 0)),
        out_shape=jax.ShapeDtypeStruct((nu, ch2), jnp.float32),
    )(x_up, xk0, xk1, w0, w1, wa, wb, c0, g0, e0, w1w, c1, g1, e1)


def _stage(p_up, x_up, b_up, p_down, x_down, b_down,
           w_l0, c_l0, g_l0, e_l0, w_l1, c_l1, g_l1, e_l1, chunk):
    nu = p_up.shape[0]
    nd = p_down.shape[0]
    nj = nd // chunk
    cup = x_up.shape[1]
    zu = jnp.zeros((nu, 1), jnp.float32)
    pu8 = jnp.concatenate(
        [p_up * -2.0, jnp.sum(p_up * p_up, axis=1, keepdims=True),
         jnp.ones((nu, 1), jnp.float32), zu,
         b_up.astype(jnp.float32).reshape(nu, 1), zu], axis=1)
    bdow = b_down.astype(jnp.float32).reshape(1, nd)
    pd_ex = jnp.concatenate(
        [p_down.T, jnp.ones((1, nd), jnp.float32),
         jnp.sum(p_down * p_down, axis=1).reshape(1, nd),
         bdow, jnp.zeros((2, nd), jnp.float32)], axis=0)
    pd3d = pd_ex.reshape(8, nj, chunk).swapaxes(0, 1)
    bdf = bdow[0, ::chunk].reshape(1, nj)
    bdl = bdow[0, chunk - 1::chunk].reshape(1, nj)
    i0, i1, w0, w1 = _knn(pu8, pd3d, bdf, bdl, nd, chunk)
    xk0, xk1 = _gather_pairs(x_down, i0.reshape(nu), i1.reshape(nu))
    wa = w_l0[:, :cup]
    wb = w_l0[:, cup:]
    row = lambda v: v.reshape(1, -1)
    return _mlp(x_up, xk0, xk1, w0, w1, wa, wb,
                row(c_l0), row(g_l0), row(e_l0),
                w_l1, row(c_l1), row(g_l1), row(e_l1))


def kernel(p0, x0, b0, p1, x1, b1, p2, x2, b2, p3, x3, b3,
           W0_0, c0_0, g0_0, e0_0, W0_1, c0_1, g0_1, e0_1,
           W1_0, c1_0, g1_0, e1_0, W1_1, c1_1, g1_1, e1_1,
           W2_0, c2_0, g2_0, e2_0, W2_1, c2_1, g2_1, e2_1):
    x2n = _stage(p2, x2, b2, p3, x3, b3,
                 W2_0, c2_0, g2_0, e2_0, W2_1, c2_1, g2_1, e2_1, chunk=128)
    x1n = _stage(p1, x1, b1, p2, x2n, b2,
                 W1_0, c1_0, g1_0, e1_0, W1_1, c1_1, g1_1, e1_1, chunk=256)
    x0n = _stage(p0, x0, b0, p1, x1n, b1,
                 W0_0, c0_0, g0_0, e0_0, W0_1, c0_1, g0_1, e0_1, chunk=1024)
    return x0n
